# baseline (speedup 1.0000x reference)
import jax
import jax.numpy as jnp
from jax import lax
from jax.experimental import pallas as pl
from jax.experimental.pallas import tpu as pltpu
from jax.experimental.pallas import tpu_sc as plsc

V = 1000000
D = 64
B = 16384
NC = 2
NS = 16
NW = NC * NS
BPW = B // NW
L = 16
C = 256


def _glove_body(ctx_hbm, tgt_hbm, wt_hbm, bt_hbm, wc_hbm, bc_hbm, out_hbm,
                tv, cv, pvt, pvc, wtb, wcb, btv, bcv, outv, sem):
    wid = lax.axis_index("s") * NC + lax.axis_index("c")
    base = wid * BPW

    pltpu.sync_copy(tgt_hbm.at[pl.ds(base, BPW)], tv)
    pltpu.sync_copy(ctx_hbm.at[pl.ds(base, BPW)], cv)

    def mk_pairs(i, carry):
        sl = pl.ds(i * L, L)
        pvt[sl] = lax.shift_right_logical(tv[sl], 1)
        pvc[sl] = lax.shift_right_logical(cv[sl], 1)
        return carry

    lax.fori_loop(0, BPW // L, mk_pairs, 0, unroll=4)

    cp_bt = pltpu.async_copy(bt_hbm.at[tv], btv, sem)
    cp_bc = pltpu.async_copy(bc_hbm.at[cv], bcv, sem)

    lane = lax.iota(jnp.int32, L)

    for c in range(BPW // C):
        off = c * C
        cp_wt = pltpu.async_copy(wt_hbm.at[pvt.at[pl.ds(off, C)]], wtb, sem)
        cp_wc = pltpu.async_copy(wc_hbm.at[pvc.at[pl.ds(off, C)]], wcb, sem)
        cp_wt.wait()
        cp_wc.wait()

        def group(g, carry):
            gsl = pl.ds(off + g * L, L)
            rows16 = lane + g * L
            hofft = lax.shift_left(jnp.bitwise_and(tv[gsl], 1), 6)
            hoffc = lax.shift_left(jnp.bitwise_and(cv[gsl], 1), 6)

            def dstep(d, acc):
                a = plsc.load_gather(wtb, [rows16, hofft + d])
                b = plsc.load_gather(wcb, [rows16, hoffc + d])
                return acc + a * b

            acc = lax.fori_loop(0, D, dstep, jnp.zeros((L,), jnp.float32),
                                unroll=16)
            outv[gsl] = acc
            return carry

        lax.fori_loop(0, C // L, group, 0)

    cp_bt.wait()
    cp_bc.wait()

    def addbias(i, carry):
        sl = pl.ds(i * L, L)
        outv[sl] = outv[sl] + btv[sl] + bcv[sl]
        return carry

    lax.fori_loop(0, BPW // L, addbias, 0, unroll=4)

    pltpu.sync_copy(outv, out_hbm.at[pl.ds(base, BPW)])


@jax.jit
def _glove_sc(context_input, target_input, W_target2, b_target_flat,
              W_context2, b_context_flat):
    mesh = plsc.VectorSubcoreMesh(core_axis_name="c", subcore_axis_name="s")
    return pl.kernel(
        _glove_body,
        mesh=mesh,
        compiler_params=pltpu.CompilerParams(
            needs_layout_passes=False, use_tc_tiling_on_sc=True),
        out_type=jax.ShapeDtypeStruct((B,), jnp.float32),
        scratch_types=[
            pltpu.VMEM((BPW,), jnp.int32),
            pltpu.VMEM((BPW,), jnp.int32),
            pltpu.VMEM((BPW,), jnp.int32),
            pltpu.VMEM((BPW,), jnp.int32),
            pltpu.VMEM((C, 2 * D), jnp.float32),
            pltpu.VMEM((C, 2 * D), jnp.float32),
            pltpu.VMEM((BPW,), jnp.float32),
            pltpu.VMEM((BPW,), jnp.float32),
            pltpu.VMEM((BPW,), jnp.float32),
            pltpu.SemaphoreType.DMA,
        ],
    )(context_input, target_input, W_target2, b_target_flat,
      W_context2, b_context_flat)


def kernel(context_input, target_input, W_target, b_target, W_context,
           b_context):
    return _glove_sc(
        context_input.astype(jnp.int32),
        target_input.astype(jnp.int32),
        jnp.reshape(W_target, (V // 2, 2 * D)),
        jnp.reshape(b_target, (V,)),
        jnp.reshape(W_context, (V // 2, 2 * D)),
        jnp.reshape(b_context, (V,)),
    )


# speedup vs baseline: 1.4425x; 1.4425x over previous
import functools

import jax
import jax.numpy as jnp
from jax import lax
from jax.experimental import pallas as pl
from jax.experimental.pallas import tpu as pltpu
from jax.experimental.pallas import tpu_sc as plsc

V = 1000000
D = 64
B = 16384
NC = 2
NS = 16
NW = NC * NS
BPW = B // NW
L = 16
C = 256

TBLK = 2048
TGRID = 245
S = TBLK * TGRID


def _repack_body(lo_ref, hi_ref, out_ref):
    out_ref[...] = jnp.concatenate(
        [jnp.transpose(lo_ref[...]), jnp.transpose(hi_ref[...])], axis=1)


@jax.jit
def _repack(wT):
    return pl.pallas_call(
        _repack_body,
        grid=(TGRID,),
        in_specs=[
            pl.BlockSpec((D, TBLK), lambda i: (0, i)),
            pl.BlockSpec((D, TBLK),
                         lambda i: (0, jnp.minimum(i + TGRID, V // TBLK))),
        ],
        out_specs=pl.BlockSpec((TBLK, 2 * D), lambda i: (i, 0)),
        out_shape=jax.ShapeDtypeStruct((S, 2 * D), jnp.float32),
    )(wT, wT)


def _glove_body(ctx_hbm, tgt_hbm, wt_hbm, bt_hbm, wc_hbm, bc_hbm, out_hbm,
                tv, cv, pvt, pvc, wtb, wcb, btv, bcv, outv, sem):
    wid = lax.axis_index("s") * NC + lax.axis_index("c")
    base = wid * BPW

    pltpu.sync_copy(tgt_hbm.at[pl.ds(base, BPW)], tv)
    pltpu.sync_copy(ctx_hbm.at[pl.ds(base, BPW)], cv)

    def mk_pairs(i, carry):
        sl = pl.ds(i * L, L)
        t16 = tv[sl]
        c16 = cv[sl]
        pvt[sl] = t16 - jnp.where(t16 >= S, S, 0)
        pvc[sl] = c16 - jnp.where(c16 >= S, S, 0)
        return carry

    lax.fori_loop(0, BPW // L, mk_pairs, 0, unroll=4)

    cp_bt = pltpu.async_copy(bt_hbm.at[tv], btv, sem)
    cp_bc = pltpu.async_copy(bc_hbm.at[cv], bcv, sem)

    lane = lax.iota(jnp.int32, L)

    for c in range(BPW // C):
        off = c * C
        cp_wt = pltpu.async_copy(wt_hbm.at[pvt.at[pl.ds(off, C)]], wtb, sem)
        cp_wc = pltpu.async_copy(wc_hbm.at[pvc.at[pl.ds(off, C)]], wcb, sem)
        cp_wt.wait()
        cp_wc.wait()

        def group(g, carry):
            gsl = pl.ds(off + g * L, L)
            rows16 = lane + g * L
            hofft = jnp.where(tv[gsl] >= S, D, 0)
            hoffc = jnp.where(cv[gsl] >= S, D, 0)

            def dstep(d, acc):
                a = plsc.load_gather(wtb, [rows16, hofft + d])
                b = plsc.load_gather(wcb, [rows16, hoffc + d])
                return acc + a * b

            acc = lax.fori_loop(0, D, dstep, jnp.zeros((L,), jnp.float32),
                                unroll=16)
            outv[gsl] = acc
            return carry

        lax.fori_loop(0, C // L, group, 0)

    cp_bt.wait()
    cp_bc.wait()

    def addbias(i, carry):
        sl = pl.ds(i * L, L)
        outv[sl] = outv[sl] + btv[sl] + bcv[sl]
        return carry

    lax.fori_loop(0, BPW // L, addbias, 0, unroll=4)

    pltpu.sync_copy(outv, out_hbm.at[pl.ds(base, BPW)])


@jax.jit
def _glove_sc(context_input, target_input, W_target2, b_target_flat,
              W_context2, b_context_flat):
    mesh = plsc.VectorSubcoreMesh(core_axis_name="c", subcore_axis_name="s")
    return pl.kernel(
        _glove_body,
        mesh=mesh,
        compiler_params=pltpu.CompilerParams(
            needs_layout_passes=False, use_tc_tiling_on_sc=True),
        out_type=jax.ShapeDtypeStruct((B,), jnp.float32),
        scratch_types=[
            pltpu.VMEM((BPW,), jnp.int32),
            pltpu.VMEM((BPW,), jnp.int32),
            pltpu.VMEM((BPW,), jnp.int32),
            pltpu.VMEM((BPW,), jnp.int32),
            pltpu.VMEM((C, 2 * D), jnp.float32),
            pltpu.VMEM((C, 2 * D), jnp.float32),
            pltpu.VMEM((BPW,), jnp.float32),
            pltpu.VMEM((BPW,), jnp.float32),
            pltpu.VMEM((BPW,), jnp.float32),
            pltpu.SemaphoreType.DMA,
        ],
    )(context_input, target_input, W_target2, b_target_flat,
      W_context2, b_context_flat)


def kernel(context_input, target_input, W_target, b_target, W_context,
           b_context):
    return _glove_sc(
        context_input.astype(jnp.int32),
        target_input.astype(jnp.int32),
        _repack(jnp.transpose(W_target)),
        jnp.reshape(b_target, (V,)),
        _repack(jnp.transpose(W_context)),
        jnp.reshape(b_context, (V,)),
    )


# speedup vs baseline: 2.1646x; 1.5006x over previous
import functools

import jax
import jax.numpy as jnp
from jax import lax
from jax.experimental import pallas as pl
from jax.experimental.pallas import tpu as pltpu
from jax.experimental.pallas import tpu_sc as plsc

V = 1000000
D = 64
B = 16384
NC = 2
NS = 16
NW = NC * NS
BPW = B // NW
L = 16
C = 256

TBLK = 4096
TGRID = 123
S = TBLK * TGRID


def _repack_body(lo_ref, hi_ref, out_ref):
    x = jnp.concatenate([lo_ref[...], hi_ref[...]], axis=0)
    eye = jnp.eye(2 * D, dtype=jnp.float32)
    out_ref[...] = lax.dot_general(
        x, eye, (((0,), (0,)), ((), ())),
        preferred_element_type=jnp.float32)


@jax.jit
def _repack(wT):
    return pl.pallas_call(
        _repack_body,
        grid=(TGRID,),
        in_specs=[
            pl.BlockSpec((D, TBLK), lambda i: (0, i)),
            pl.BlockSpec((D, TBLK),
                         lambda i: (0, jnp.minimum(i + TGRID, V // TBLK))),
        ],
        out_specs=pl.BlockSpec((TBLK, 2 * D), lambda i: (i, 0)),
        out_shape=jax.ShapeDtypeStruct((S, 2 * D), jnp.float32),
    )(wT, wT)


def _glove_body(ctx_hbm, tgt_hbm, wt_hbm, bt_hbm, wc_hbm, bc_hbm, out_hbm,
                tv, cv, pvt, pvc, wtb, wcb, btv, bcv, outv, sem):
    wid = lax.axis_index("s") * NC + lax.axis_index("c")
    base = wid * BPW

    pltpu.sync_copy(tgt_hbm.at[pl.ds(base, BPW)], tv)
    pltpu.sync_copy(ctx_hbm.at[pl.ds(base, BPW)], cv)

    def mk_pairs(i, carry):
        sl = pl.ds(i * L, L)
        t16 = tv[sl]
        c16 = cv[sl]
        pvt[sl] = t16 - jnp.where(t16 >= S, S, 0)
        pvc[sl] = c16 - jnp.where(c16 >= S, S, 0)
        return carry

    lax.fori_loop(0, BPW // L, mk_pairs, 0, unroll=4)

    cp_bt = pltpu.async_copy(bt_hbm.at[tv], btv, sem)
    cp_bc = pltpu.async_copy(bc_hbm.at[cv], bcv, sem)

    lane = lax.iota(jnp.int32, L)

    for c in range(BPW // C):
        off = c * C
        cp_wt = pltpu.async_copy(wt_hbm.at[pvt.at[pl.ds(off, C)]], wtb, sem)
        cp_wc = pltpu.async_copy(wc_hbm.at[pvc.at[pl.ds(off, C)]], wcb, sem)
        cp_wt.wait()
        cp_wc.wait()

        def group(g, carry):
            gsl = pl.ds(off + g * L, L)
            rows16 = lane + g * L
            hofft = jnp.where(tv[gsl] >= S, D, 0)
            hoffc = jnp.where(cv[gsl] >= S, D, 0)

            def dstep(d, acc):
                a = plsc.load_gather(wtb, [rows16, hofft + d])
                b = plsc.load_gather(wcb, [rows16, hoffc + d])
                return acc + a * b

            acc = lax.fori_loop(0, D, dstep, jnp.zeros((L,), jnp.float32),
                                unroll=16)
            outv[gsl] = acc
            return carry

        lax.fori_loop(0, C // L, group, 0)

    cp_bt.wait()
    cp_bc.wait()

    def addbias(i, carry):
        sl = pl.ds(i * L, L)
        outv[sl] = outv[sl] + btv[sl] + bcv[sl]
        return carry

    lax.fori_loop(0, BPW // L, addbias, 0, unroll=4)

    pltpu.sync_copy(outv, out_hbm.at[pl.ds(base, BPW)])


@jax.jit
def _glove_sc(context_input, target_input, W_target2, b_target_flat,
              W_context2, b_context_flat):
    mesh = plsc.VectorSubcoreMesh(core_axis_name="c", subcore_axis_name="s")
    return pl.kernel(
        _glove_body,
        mesh=mesh,
        compiler_params=pltpu.CompilerParams(
            needs_layout_passes=False, use_tc_tiling_on_sc=True),
        out_type=jax.ShapeDtypeStruct((B,), jnp.float32),
        scratch_types=[
            pltpu.VMEM((BPW,), jnp.int32),
            pltpu.VMEM((BPW,), jnp.int32),
            pltpu.VMEM((BPW,), jnp.int32),
            pltpu.VMEM((BPW,), jnp.int32),
            pltpu.VMEM((C, 2 * D), jnp.float32),
            pltpu.VMEM((C, 2 * D), jnp.float32),
            pltpu.VMEM((BPW,), jnp.float32),
            pltpu.VMEM((BPW,), jnp.float32),
            pltpu.VMEM((BPW,), jnp.float32),
            pltpu.SemaphoreType.DMA,
        ],
    )(context_input, target_input, W_target2, b_target_flat,
      W_context2, b_context_flat)


def kernel(context_input, target_input, W_target, b_target, W_context,
           b_context):
    return _glove_sc(
        context_input.astype(jnp.int32),
        target_input.astype(jnp.int32),
        _repack(jnp.transpose(W_target)),
        jnp.reshape(b_target, (V,)),
        _repack(jnp.transpose(W_context)),
        jnp.reshape(b_context, (V,)),
    )


# speedup vs baseline: 2.4030x; 1.1101x over previous
import functools

import jax
import jax.numpy as jnp
from jax import lax
from jax.experimental import pallas as pl
from jax.experimental.pallas import tpu as pltpu
from jax.experimental.pallas import tpu_sc as plsc

V = 1000000
D = 64
B = 16384
NC = 2
NS = 16
NW = NC * NS
BPW = B // NW
L = 16
C = 256

TBLK = 8192
TGRID = 62
S = TBLK * TGRID


def _repack_body(lo_ref, hi_ref, out_ref):
    x = jnp.concatenate([lo_ref[...], hi_ref[...]], axis=0)
    eye = jnp.eye(2 * D, dtype=jnp.float32)
    out_ref[...] = lax.dot_general(
        x, eye, (((0,), (0,)), ((), ())),
        preferred_element_type=jnp.float32)


@jax.jit
def _repack(wT):
    return pl.pallas_call(
        _repack_body,
        grid=(TGRID,),
        in_specs=[
            pl.BlockSpec((D, TBLK), lambda i: (0, i)),
            pl.BlockSpec((D, TBLK),
                         lambda i: (0, jnp.minimum(i + TGRID, V // TBLK))),
        ],
        out_specs=pl.BlockSpec((TBLK, 2 * D), lambda i: (i, 0)),
        out_shape=jax.ShapeDtypeStruct((S, 2 * D), jnp.float32),
    )(wT, wT)


def _glove_body(ctx_hbm, tgt_hbm, wt_hbm, bt_hbm, wc_hbm, bc_hbm, out_hbm,
                tv, cv, pvt, pvc, wtb, wcb, btv, bcv, outv, sem):
    wid = lax.axis_index("s") * NC + lax.axis_index("c")
    base = wid * BPW

    pltpu.sync_copy(tgt_hbm.at[pl.ds(base, BPW)], tv)
    pltpu.sync_copy(ctx_hbm.at[pl.ds(base, BPW)], cv)

    def mk_pairs(i, carry):
        sl = pl.ds(i * L, L)
        t16 = tv[sl]
        c16 = cv[sl]
        pvt[sl] = t16 - jnp.where(t16 >= S, S, 0)
        pvc[sl] = c16 - jnp.where(c16 >= S, S, 0)
        return carry

    lax.fori_loop(0, BPW // L, mk_pairs, 0, unroll=4)

    cp_bt = pltpu.async_copy(bt_hbm.at[tv], btv, sem)
    cp_bc = pltpu.async_copy(bc_hbm.at[cv], bcv, sem)

    lane = lax.iota(jnp.int32, L)

    for c in range(BPW // C):
        off = c * C
        cp_wt = pltpu.async_copy(wt_hbm.at[pvt.at[pl.ds(off, C)]], wtb, sem)
        cp_wc = pltpu.async_copy(wc_hbm.at[pvc.at[pl.ds(off, C)]], wcb, sem)
        cp_wt.wait()
        cp_wc.wait()

        def group(g, carry):
            gsl = pl.ds(off + g * L, L)
            rows16 = lane + g * L
            hofft = jnp.where(tv[gsl] >= S, D, 0)
            hoffc = jnp.where(cv[gsl] >= S, D, 0)

            def dstep(d, acc):
                a = plsc.load_gather(wtb, [rows16, hofft + d])
                b = plsc.load_gather(wcb, [rows16, hoffc + d])
                return acc + a * b

            acc = lax.fori_loop(0, D, dstep, jnp.zeros((L,), jnp.float32),
                                unroll=16)
            outv[gsl] = acc
            return carry

        lax.fori_loop(0, C // L, group, 0)

    cp_bt.wait()
    cp_bc.wait()

    def addbias(i, carry):
        sl = pl.ds(i * L, L)
        outv[sl] = outv[sl] + btv[sl] + bcv[sl]
        return carry

    lax.fori_loop(0, BPW // L, addbias, 0, unroll=4)

    pltpu.sync_copy(outv, out_hbm.at[pl.ds(base, BPW)])


@jax.jit
def _glove_sc(context_input, target_input, W_target2, b_target_flat,
              W_context2, b_context_flat):
    mesh = plsc.VectorSubcoreMesh(core_axis_name="c", subcore_axis_name="s")
    return pl.kernel(
        _glove_body,
        mesh=mesh,
        compiler_params=pltpu.CompilerParams(
            needs_layout_passes=False, use_tc_tiling_on_sc=True),
        out_type=jax.ShapeDtypeStruct((B,), jnp.float32),
        scratch_types=[
            pltpu.VMEM((BPW,), jnp.int32),
            pltpu.VMEM((BPW,), jnp.int32),
            pltpu.VMEM((BPW,), jnp.int32),
            pltpu.VMEM((BPW,), jnp.int32),
            pltpu.VMEM((C, 2 * D), jnp.float32),
            pltpu.VMEM((C, 2 * D), jnp.float32),
            pltpu.VMEM((BPW,), jnp.float32),
            pltpu.VMEM((BPW,), jnp.float32),
            pltpu.VMEM((BPW,), jnp.float32),
            pltpu.SemaphoreType.DMA,
        ],
    )(context_input, target_input, W_target2, b_target_flat,
      W_context2, b_context_flat)


def kernel(context_input, target_input, W_target, b_target, W_context,
           b_context):
    return _glove_sc(
        context_input.astype(jnp.int32),
        target_input.astype(jnp.int32),
        _repack(jnp.transpose(W_target)),
        jnp.reshape(b_target, (V,)),
        _repack(jnp.transpose(W_context)),
        jnp.reshape(b_context, (V,)),
    )


# speedup vs baseline: 2.4535x; 1.0210x over previous
import functools

import jax
import jax.numpy as jnp
from jax import lax
from jax.experimental import pallas as pl
from jax.experimental.pallas import tpu as pltpu
from jax.experimental.pallas import tpu_sc as plsc

V = 1000000
D = 64
B = 16384
NC = 2
NS = 16
NW = NC * NS
BPW = B // NW
L = 16
C = 256

TBLK = 16384
TGRID = 31
S = TBLK * TGRID


def _repack_body(lo_ref, hi_ref, out_ref):
    x = jnp.concatenate([lo_ref[...], hi_ref[...]], axis=0)
    eye = jnp.eye(2 * D, dtype=jnp.float32)
    out_ref[...] = lax.dot_general(
        x, eye, (((0,), (0,)), ((), ())),
        preferred_element_type=jnp.float32)


@jax.jit
def _repack(wT):
    return pl.pallas_call(
        _repack_body,
        grid=(TGRID,),
        in_specs=[
            pl.BlockSpec((D, TBLK), lambda i: (0, i)),
            pl.BlockSpec((D, TBLK),
                         lambda i: (0, jnp.minimum(i + TGRID, V // TBLK))),
        ],
        out_specs=pl.BlockSpec((TBLK, 2 * D), lambda i: (i, 0)),
        out_shape=jax.ShapeDtypeStruct((S, 2 * D), jnp.float32),
    )(wT, wT)


def _glove_body(ctx_hbm, tgt_hbm, wt_hbm, bt_hbm, wc_hbm, bc_hbm, out_hbm,
                tv, cv, pvt, pvc, wtb, wcb, btv, bcv, outv, sem):
    wid = lax.axis_index("s") * NC + lax.axis_index("c")
    base = wid * BPW

    pltpu.sync_copy(tgt_hbm.at[pl.ds(base, BPW)], tv)
    pltpu.sync_copy(ctx_hbm.at[pl.ds(base, BPW)], cv)

    def mk_pairs(i, carry):
        sl = pl.ds(i * L, L)
        t16 = tv[sl]
        c16 = cv[sl]
        pvt[sl] = t16 - jnp.where(t16 >= S, S, 0)
        pvc[sl] = c16 - jnp.where(c16 >= S, S, 0)
        return carry

    lax.fori_loop(0, BPW // L, mk_pairs, 0, unroll=4)

    cp_bt = pltpu.async_copy(bt_hbm.at[tv], btv, sem)
    cp_bc = pltpu.async_copy(bc_hbm.at[cv], bcv, sem)

    lane = lax.iota(jnp.int32, L)

    for c in range(BPW // C):
        off = c * C
        cp_wt = pltpu.async_copy(wt_hbm.at[pvt.at[pl.ds(off, C)]], wtb, sem)
        cp_wc = pltpu.async_copy(wc_hbm.at[pvc.at[pl.ds(off, C)]], wcb, sem)
        cp_wt.wait()
        cp_wc.wait()

        def group(g, carry):
            gsl = pl.ds(off + g * L, L)
            rows16 = lane + g * L
            hofft = jnp.where(tv[gsl] >= S, D, 0)
            hoffc = jnp.where(cv[gsl] >= S, D, 0)

            def dstep(d, acc):
                a = plsc.load_gather(wtb, [rows16, hofft + d])
                b = plsc.load_gather(wcb, [rows16, hoffc + d])
                return acc + a * b

            acc = lax.fori_loop(0, D, dstep, jnp.zeros((L,), jnp.float32),
                                unroll=16)
            outv[gsl] = acc
            return carry

        lax.fori_loop(0, C // L, group, 0)

    cp_bt.wait()
    cp_bc.wait()

    def addbias(i, carry):
        sl = pl.ds(i * L, L)
        outv[sl] = outv[sl] + btv[sl] + bcv[sl]
        return carry

    lax.fori_loop(0, BPW // L, addbias, 0, unroll=4)

    pltpu.sync_copy(outv, out_hbm.at[pl.ds(base, BPW)])


@jax.jit
def _glove_sc(context_input, target_input, W_target2, b_target_flat,
              W_context2, b_context_flat):
    mesh = plsc.VectorSubcoreMesh(core_axis_name="c", subcore_axis_name="s")
    return pl.kernel(
        _glove_body,
        mesh=mesh,
        compiler_params=pltpu.CompilerParams(
            needs_layout_passes=False, use_tc_tiling_on_sc=True),
        out_type=jax.ShapeDtypeStruct((B,), jnp.float32),
        scratch_types=[
            pltpu.VMEM((BPW,), jnp.int32),
            pltpu.VMEM((BPW,), jnp.int32),
            pltpu.VMEM((BPW,), jnp.int32),
            pltpu.VMEM((BPW,), jnp.int32),
            pltpu.VMEM((C, 2 * D), jnp.float32),
            pltpu.VMEM((C, 2 * D), jnp.float32),
            pltpu.VMEM((BPW,), jnp.float32),
            pltpu.VMEM((BPW,), jnp.float32),
            pltpu.VMEM((BPW,), jnp.float32),
            pltpu.SemaphoreType.DMA,
        ],
    )(context_input, target_input, W_target2, b_target_flat,
      W_context2, b_context_flat)


def kernel(context_input, target_input, W_target, b_target, W_context,
           b_context):
    return _glove_sc(
        context_input.astype(jnp.int32),
        target_input.astype(jnp.int32),
        _repack(jnp.transpose(W_target)),
        jnp.reshape(b_target, (V,)),
        _repack(jnp.transpose(W_context)),
        jnp.reshape(b_context, (V,)),
    )


# speedup vs baseline: 2.4696x; 1.0066x over previous
import jax
import jax.numpy as jnp
from jax import lax
from jax.experimental import pallas as pl
from jax.experimental.pallas import tpu as pltpu
from jax.experimental.pallas import tpu_sc as plsc

V = 1000000
D = 64
B = 16384
NC = 2
NS = 16
NW = NC * NS
BPW = B // NW
L = 16
C = 128
NCH = BPW // C

TBLK = 16384
TGRID = 31
S = TBLK * TGRID


def _repack_body(lo_ref, hi_ref, out_ref):
    x = jnp.concatenate([lo_ref[...], hi_ref[...]], axis=0)
    eye = jnp.eye(2 * D, dtype=jnp.float32)
    out_ref[...] = lax.dot_general(
        x, eye, (((0,), (0,)), ((), ())),
        preferred_element_type=jnp.float32)


@jax.jit
def _repack(wT):
    return pl.pallas_call(
        _repack_body,
        grid=(TGRID,),
        in_specs=[
            pl.BlockSpec((D, TBLK), lambda i: (0, i)),
            pl.BlockSpec((D, TBLK),
                         lambda i: (0, jnp.minimum(i + TGRID, V // TBLK))),
        ],
        out_specs=pl.BlockSpec((TBLK, 2 * D), lambda i: (i, 0)),
        out_shape=jax.ShapeDtypeStruct((S, 2 * D), jnp.float32),
    )(wT, wT)


def _split_pack_rows(iv, pv):
    def body(i, carry):
        sl = pl.ds(i * L, L)
        v16 = iv[sl]
        pv[sl] = v16 - jnp.where(v16 >= S, S, 0)
        return carry

    lax.fori_loop(0, BPW // L, body, 0, unroll=4)


def _stage_body(tgt_hbm, pt_hbm, stage_hbm, tv, pvt, rows, sem):
    wid = lax.axis_index("s") * NC + lax.axis_index("c")
    base = wid * BPW
    pltpu.sync_copy(tgt_hbm.at[pl.ds(base, BPW)], tv)
    _split_pack_rows(tv, pvt)
    pltpu.async_copy(pt_hbm.at[pvt], rows, sem).wait()
    pltpu.sync_copy(rows, stage_hbm.at[pl.ds(base, BPW)])


@jax.jit
def _stage_target(target_input, P_target):
    mesh = plsc.VectorSubcoreMesh(core_axis_name="c", subcore_axis_name="s")
    return pl.kernel(
        _stage_body,
        mesh=mesh,
        compiler_params=pltpu.CompilerParams(
            needs_layout_passes=False, use_tc_tiling_on_sc=True),
        out_type=jax.ShapeDtypeStruct((B, 2 * D), jnp.float32),
        scratch_types=[
            pltpu.VMEM((BPW,), jnp.int32),
            pltpu.VMEM((BPW,), jnp.int32),
            pltpu.VMEM((BPW, 2 * D), jnp.float32),
            pltpu.SemaphoreType.DMA,
        ],
    )(target_input, P_target)


def _glove_body(ctx_hbm, tgt_hbm, pc_hbm, stage_hbm, bt_hbm, bc_hbm, out_hbm,
                tv, cv, pvc, wtb, wcb, btv, bcv, outv, sem0, sem1, bsem):
    wid = lax.axis_index("s") * NC + lax.axis_index("c")
    base = wid * BPW

    pltpu.sync_copy(tgt_hbm.at[pl.ds(base, BPW)], tv)
    pltpu.sync_copy(ctx_hbm.at[pl.ds(base, BPW)], cv)
    _split_pack_rows(cv, pvc)

    cp_bt = pltpu.async_copy(bt_hbm.at[tv], btv, bsem)
    cp_bc = pltpu.async_copy(bc_hbm.at[cv], bcv, bsem)

    lane = lax.iota(jnp.int32, L)

    sems = (sem0, sem1)

    def fire(c, buf):
        off = base + c * C
        pltpu.async_copy(stage_hbm.at[pl.ds(off, C)], wtb.at[buf], sems[buf])
        pltpu.async_copy(pc_hbm.at[pvc.at[pl.ds(c * C, C)]],
                         wcb.at[buf], sems[buf])

    def drain(buf):
        pltpu.make_async_copy(stage_hbm.at[pl.ds(base, C)],
                              wtb.at[buf], sems[buf]).wait()
        pltpu.make_async_copy(stage_hbm.at[pl.ds(base, C)],
                              wcb.at[buf], sems[buf]).wait()

    fire(0, 0)
    for c in range(NCH):
        buf = c % 2
        if c + 1 < NCH:
            fire(c + 1, 1 - buf)
        drain(buf)
        off = c * C

        def group(g, carry):
            gsl = pl.ds(off + g * L, L)
            rows16 = lane + g * L
            hofft = jnp.where(tv[gsl] >= S, D, 0)
            hoffc = jnp.where(cv[gsl] >= S, D, 0)

            def dstep(d, acc):
                a = plsc.load_gather(wtb.at[buf], [rows16, hofft + d])
                b = plsc.load_gather(wcb.at[buf], [rows16, hoffc + d])
                return acc + a * b

            acc = lax.fori_loop(0, D, dstep, jnp.zeros((L,), jnp.float32),
                                unroll=16)
            outv[gsl] = acc
            return carry

        lax.fori_loop(0, C // L, group, 0)

    cp_bt.wait()
    cp_bc.wait()

    def addbias(i, carry):
        sl = pl.ds(i * L, L)
        outv[sl] = outv[sl] + btv[sl] + bcv[sl]
        return carry

    lax.fori_loop(0, BPW // L, addbias, 0, unroll=4)

    pltpu.sync_copy(outv, out_hbm.at[pl.ds(base, BPW)])


@jax.jit
def _glove_sc(context_input, target_input, P_context, staged_target,
              b_target_flat, b_context_flat):
    mesh = plsc.VectorSubcoreMesh(core_axis_name="c", subcore_axis_name="s")
    return pl.kernel(
        _glove_body,
        mesh=mesh,
        compiler_params=pltpu.CompilerParams(
            needs_layout_passes=False, use_tc_tiling_on_sc=True),
        out_type=jax.ShapeDtypeStruct((B,), jnp.float32),
        scratch_types=[
            pltpu.VMEM((BPW,), jnp.int32),
            pltpu.VMEM((BPW,), jnp.int32),
            pltpu.VMEM((BPW,), jnp.int32),
            pltpu.VMEM((2, C, 2 * D), jnp.float32),
            pltpu.VMEM((2, C, 2 * D), jnp.float32),
            pltpu.VMEM((BPW,), jnp.float32),
            pltpu.VMEM((BPW,), jnp.float32),
            pltpu.VMEM((BPW,), jnp.float32),
            pltpu.SemaphoreType.DMA,
            pltpu.SemaphoreType.DMA,
            pltpu.SemaphoreType.DMA,
        ],
    )(context_input, target_input, P_context, staged_target,
      b_target_flat, b_context_flat)


def kernel(context_input, target_input, W_target, b_target, W_context,
           b_context):
    ctx = context_input.astype(jnp.int32)
    tgt = target_input.astype(jnp.int32)
    staged = _stage_target(tgt, _repack(jnp.transpose(W_target)))
    return _glove_sc(
        ctx,
        tgt,
        _repack(jnp.transpose(W_context)),
        staged,
        jnp.reshape(b_target, (V,)),
        jnp.reshape(b_context, (V,)),
    )


# speedup vs baseline: 2.8319x; 1.1467x over previous
import jax
import jax.numpy as jnp
from jax import lax
from jax.experimental import pallas as pl
from jax.experimental.pallas import tpu as pltpu
from jax.experimental.pallas import tpu_sc as plsc

V = 1000000
D = 64
B = 16384
NC = 2
NS = 16
NW = NC * NS
BPW = B // NW
L = 16
C = 128
NCH = BPW // C

TBLK = 16384
TGRID = 16
S = TBLK * TGRID
NWIN = 4


def _repack_body(w0_ref, w1_ref, w2_ref, w3_ref, out_ref):
    x = jnp.concatenate(
        [w0_ref[...], w1_ref[...], w2_ref[...], w3_ref[...]], axis=0)
    eye = jnp.eye(4 * D, dtype=jnp.float32)
    y = lax.dot_general(x, eye, (((0,), (0,)), ((), ())),
                        preferred_element_type=jnp.float32)

    def bf16_bits(f):
        r = lax.bitcast_convert_type(f, jnp.int32)
        r = r + 32767 + jnp.bitwise_and(lax.shift_right_arithmetic(r, 16), 1)
        return jnp.bitwise_and(lax.shift_right_arithmetic(r, 16), 65535)

    lo = bf16_bits(y[:, : 2 * D])
    hi = bf16_bits(y[:, 2 * D :])
    out_ref[...] = jnp.bitwise_or(lo, lax.shift_left(hi, 16))


@jax.jit
def _repack(wT):
    return pl.pallas_call(
        _repack_body,
        grid=(TGRID,),
        in_specs=[
            pl.BlockSpec(
                (D, TBLK),
                lambda i, k=k: (0, jnp.minimum(i + k * TGRID, V // TBLK)))
            for k in range(NWIN)
        ],
        out_specs=pl.BlockSpec((TBLK, 2 * D), lambda i: (i, 0)),
        out_shape=jax.ShapeDtypeStruct((S, 2 * D), jnp.int32),
    )(wT, wT, wT, wT)


def _split_pack_rows(iv, pv):
    def body(i, carry):
        sl = pl.ds(i * L, L)
        v16 = iv[sl]
        sub = (jnp.where(v16 >= S, S, 0) + jnp.where(v16 >= 2 * S, S, 0)
               + jnp.where(v16 >= 3 * S, S, 0))
        pv[sl] = v16 - sub
        return carry

    lax.fori_loop(0, BPW // L, body, 0, unroll=4)


def _stage_body(tgt_hbm, pt_hbm, stage_hbm, tv, pvt, rows, sem):
    wid = lax.axis_index("s") * NC + lax.axis_index("c")
    base = wid * BPW
    pltpu.sync_copy(tgt_hbm.at[pl.ds(base, BPW)], tv)
    _split_pack_rows(tv, pvt)
    pltpu.async_copy(pt_hbm.at[pvt], rows, sem).wait()
    pltpu.sync_copy(rows, stage_hbm.at[pl.ds(base, BPW)])


@jax.jit
def _stage_target(target_input, P_target):
    mesh = plsc.VectorSubcoreMesh(core_axis_name="c", subcore_axis_name="s")
    return pl.kernel(
        _stage_body,
        mesh=mesh,
        compiler_params=pltpu.CompilerParams(
            needs_layout_passes=False, use_tc_tiling_on_sc=True),
        out_type=jax.ShapeDtypeStruct((B, 2 * D), jnp.int32),
        scratch_types=[
            pltpu.VMEM((BPW,), jnp.int32),
            pltpu.VMEM((BPW,), jnp.int32),
            pltpu.VMEM((BPW, 2 * D), jnp.int32),
            pltpu.SemaphoreType.DMA,
        ],
    )(target_input, P_target)


def _glove_body(ctx_hbm, tgt_hbm, pc_hbm, stage_hbm, bt_hbm, bc_hbm, out_hbm,
                tv, cv, pvc, wtb, wcb, btv, bcv, outv, sem0, sem1, bsem):
    wid = lax.axis_index("s") * NC + lax.axis_index("c")
    base = wid * BPW

    pltpu.sync_copy(tgt_hbm.at[pl.ds(base, BPW)], tv)
    pltpu.sync_copy(ctx_hbm.at[pl.ds(base, BPW)], cv)
    _split_pack_rows(cv, pvc)

    cp_bt = pltpu.async_copy(bt_hbm.at[tv], btv, bsem)
    cp_bc = pltpu.async_copy(bc_hbm.at[cv], bcv, bsem)

    lane = lax.iota(jnp.int32, L)

    sems = (sem0, sem1)

    def fire(c, buf):
        off = base + c * C
        pltpu.async_copy(stage_hbm.at[pl.ds(off, C)], wtb.at[buf], sems[buf])
        pltpu.async_copy(pc_hbm.at[pvc.at[pl.ds(c * C, C)]],
                         wcb.at[buf], sems[buf])

    def drain(buf):
        pltpu.make_async_copy(stage_hbm.at[pl.ds(base, C)],
                              wtb.at[buf], sems[buf]).wait()
        pltpu.make_async_copy(stage_hbm.at[pl.ds(base, C)],
                              wcb.at[buf], sems[buf]).wait()

    fire(0, 0)
    for c in range(NCH):
        buf = c % 2
        if c + 1 < NCH:
            fire(c + 1, 1 - buf)
        drain(buf)
        off = c * C

        def group(g, carry):
            gsl = pl.ds(off + g * L, L)
            rows16 = lane + g * L
            t16 = tv[gsl]
            c16 = cv[gsl]
            hofft = (jnp.where(t16 >= S, D, 0) - jnp.where(t16 >= 2 * S, D, 0)
                     + jnp.where(t16 >= 3 * S, D, 0))
            hoffc = (jnp.where(c16 >= S, D, 0) - jnp.where(c16 >= 2 * S, D, 0)
                     + jnp.where(c16 >= 3 * S, D, 0))
            mt = t16 >= 2 * S
            mc = c16 >= 2 * S
            himask = jnp.full((L,), -65536, jnp.int32)

            def dstep(d, acc):
                a = plsc.load_gather(wtb.at[buf], [rows16, hofft + d])
                b = plsc.load_gather(wcb.at[buf], [rows16, hoffc + d])
                av = jnp.where(
                    mt, plsc.bitcast(jnp.bitwise_and(a, himask), jnp.float32),
                    plsc.bitcast(lax.shift_left(a, 16), jnp.float32))
                bv = jnp.where(
                    mc, plsc.bitcast(jnp.bitwise_and(b, himask), jnp.float32),
                    plsc.bitcast(lax.shift_left(b, 16), jnp.float32))
                return acc + av * bv

            acc = lax.fori_loop(0, D, dstep,
                                jnp.zeros((L,), jnp.float32), unroll=16)
            outv[gsl] = acc
            return carry

        lax.fori_loop(0, C // L, group, 0)

    cp_bt.wait()
    cp_bc.wait()

    def addbias(i, carry):
        sl = pl.ds(i * L, L)
        outv[sl] = outv[sl] + btv[sl] + bcv[sl]
        return carry

    lax.fori_loop(0, BPW // L, addbias, 0, unroll=4)

    pltpu.sync_copy(outv, out_hbm.at[pl.ds(base, BPW)])


@jax.jit
def _glove_sc(context_input, target_input, P_context, staged_target,
              b_target_flat, b_context_flat):
    mesh = plsc.VectorSubcoreMesh(core_axis_name="c", subcore_axis_name="s")
    return pl.kernel(
        _glove_body,
        mesh=mesh,
        compiler_params=pltpu.CompilerParams(
            needs_layout_passes=False, use_tc_tiling_on_sc=True),
        out_type=jax.ShapeDtypeStruct((B,), jnp.float32),
        scratch_types=[
            pltpu.VMEM((BPW,), jnp.int32),
            pltpu.VMEM((BPW,), jnp.int32),
            pltpu.VMEM((BPW,), jnp.int32),
            pltpu.VMEM((2, C, 2 * D), jnp.int32),
            pltpu.VMEM((2, C, 2 * D), jnp.int32),
            pltpu.VMEM((BPW,), jnp.float32),
            pltpu.VMEM((BPW,), jnp.float32),
            pltpu.VMEM((BPW,), jnp.float32),
            pltpu.SemaphoreType.DMA,
            pltpu.SemaphoreType.DMA,
            pltpu.SemaphoreType.DMA,
        ],
    )(context_input, target_input, P_context, staged_target,
      b_target_flat, b_context_flat)


def kernel(context_input, target_input, W_target, b_target, W_context,
           b_context):
    ctx = context_input.astype(jnp.int32)
    tgt = target_input.astype(jnp.int32)
    staged = _stage_target(tgt, _repack(jnp.transpose(W_target)))
    return _glove_sc(
        ctx,
        tgt,
        _repack(jnp.transpose(W_context)),
        staged,
        jnp.reshape(b_target, (V,)),
        jnp.reshape(b_context, (V,)),
    )


# speedup vs baseline: 2.9990x; 1.0590x over previous
import jax
import jax.numpy as jnp
from jax import lax
from jax.experimental import pallas as pl
from jax.experimental.pallas import tpu as pltpu
from jax.experimental.pallas import tpu_sc as plsc

V = 1000000
D = 64
B = 16384
NC = 2
NS = 16
NW = NC * NS
BPW = B // NW
L = 16
C = 128
NCH = BPW // C

TBLK = 8192
TGRID = 32
S = TBLK * TGRID
NWIN = 4


def _pack_half(w0_ref, w1_ref, w2_ref, w3_ref, out_ref):
    x = jnp.concatenate(
        [w0_ref[...], w1_ref[...], w2_ref[...], w3_ref[...]], axis=0)
    eye = jnp.eye(4 * D, dtype=jnp.float32)
    y = lax.dot_general(x, eye, (((0,), (0,)), ((), ())),
                        preferred_element_type=jnp.float32)

    def bf16_bits(f):
        r = lax.bitcast_convert_type(f, jnp.int32)
        return jnp.bitwise_and(lax.shift_right_arithmetic(r, 16), 65535)

    lo = bf16_bits(y[:, : 2 * D])
    hi = bf16_bits(y[:, 2 * D :])
    out_ref[...] = jnp.bitwise_or(lo, lax.shift_left(hi, 16))


def _repack_body(t0, t1, t2, t3, c0, c1, c2, c3, outt_ref, outc_ref):
    _pack_half(t0, t1, t2, t3, outt_ref)
    _pack_half(c0, c1, c2, c3, outc_ref)


@jax.jit
def _repack2(wtT, wcT):
    specs = [
        pl.BlockSpec(
            (D, TBLK),
            lambda i, k=k: (0, jnp.minimum(i + k * TGRID, V // TBLK)))
        for k in range(NWIN)
    ]
    return pl.pallas_call(
        _repack_body,
        grid=(TGRID,),
        in_specs=specs + specs,
        out_specs=[pl.BlockSpec((TBLK, 2 * D), lambda i: (i, 0))] * 2,
        out_shape=[jax.ShapeDtypeStruct((S, 2 * D), jnp.int32)] * 2,
    )(wtT, wtT, wtT, wtT, wcT, wcT, wcT, wcT)


def _split_pack_rows(iv, pv):
    def body(i, carry):
        sl = pl.ds(i * L, L)
        v16 = iv[sl]
        sub = (jnp.where(v16 >= S, S, 0) + jnp.where(v16 >= 2 * S, S, 0)
               + jnp.where(v16 >= 3 * S, S, 0))
        pv[sl] = v16 - sub
        return carry

    lax.fori_loop(0, BPW // L, body, 0, unroll=4)


def _stage_body(tgt_hbm, pt_hbm, stage_hbm, tv, pvt, rows, sem):
    wid = lax.axis_index("s") * NC + lax.axis_index("c")
    base = wid * BPW
    pltpu.sync_copy(tgt_hbm.at[pl.ds(base, BPW)], tv)
    _split_pack_rows(tv, pvt)
    pltpu.async_copy(pt_hbm.at[pvt], rows, sem).wait()
    pltpu.sync_copy(rows, stage_hbm.at[pl.ds(base, BPW)])


@jax.jit
def _stage_target(target_input, P_target):
    mesh = plsc.VectorSubcoreMesh(core_axis_name="c", subcore_axis_name="s")
    return pl.kernel(
        _stage_body,
        mesh=mesh,
        compiler_params=pltpu.CompilerParams(
            needs_layout_passes=False, use_tc_tiling_on_sc=True),
        out_type=jax.ShapeDtypeStruct((B, 2 * D), jnp.int32),
        scratch_types=[
            pltpu.VMEM((BPW,), jnp.int32),
            pltpu.VMEM((BPW,), jnp.int32),
            pltpu.VMEM((BPW, 2 * D), jnp.int32),
            pltpu.SemaphoreType.DMA,
        ],
    )(target_input, P_target)


def _glove_body(ctx_hbm, tgt_hbm, pc_hbm, stage_hbm, bt_hbm, bc_hbm, out_hbm,
                tv, cv, pvc, wtb, wcb, btv, bcv, outv, sem0, sem1, bsem):
    wid = lax.axis_index("s") * NC + lax.axis_index("c")
    base = wid * BPW

    pltpu.sync_copy(tgt_hbm.at[pl.ds(base, BPW)], tv)
    pltpu.sync_copy(ctx_hbm.at[pl.ds(base, BPW)], cv)
    _split_pack_rows(cv, pvc)

    cp_bt = pltpu.async_copy(bt_hbm.at[tv], btv, bsem)
    cp_bc = pltpu.async_copy(bc_hbm.at[cv], bcv, bsem)

    lane = lax.iota(jnp.int32, L)

    sems = (sem0, sem1)

    def fire(c, buf):
        off = base + c * C
        pltpu.async_copy(stage_hbm.at[pl.ds(off, C)], wtb.at[buf], sems[buf])
        pltpu.async_copy(pc_hbm.at[pvc.at[pl.ds(c * C, C)]],
                         wcb.at[buf], sems[buf])

    def drain(buf):
        pltpu.make_async_copy(stage_hbm.at[pl.ds(base, C)],
                              wtb.at[buf], sems[buf]).wait()
        pltpu.make_async_copy(stage_hbm.at[pl.ds(base, C)],
                              wcb.at[buf], sems[buf]).wait()

    fire(0, 0)
    for c in range(NCH):
        buf = c % 2
        if c + 1 < NCH:
            fire(c + 1, 1 - buf)
        drain(buf)
        off = c * C

        def group(g, carry):
            gsl = pl.ds(off + g * L, L)
            rows16 = lane + g * L
            t16 = tv[gsl]
            c16 = cv[gsl]
            hofft = (jnp.where(t16 >= S, D, 0) - jnp.where(t16 >= 2 * S, D, 0)
                     + jnp.where(t16 >= 3 * S, D, 0))
            hoffc = (jnp.where(c16 >= S, D, 0) - jnp.where(c16 >= 2 * S, D, 0)
                     + jnp.where(c16 >= 3 * S, D, 0))
            mt = t16 >= 2 * S
            mc = c16 >= 2 * S
            himask = jnp.full((L,), -65536, jnp.int32)

            def dstep(d, acc):
                a = plsc.load_gather(wtb.at[buf], [rows16, hofft + d])
                b = plsc.load_gather(wcb.at[buf], [rows16, hoffc + d])
                av = jnp.where(
                    mt, plsc.bitcast(jnp.bitwise_and(a, himask), jnp.float32),
                    plsc.bitcast(lax.shift_left(a, 16), jnp.float32))
                bv = jnp.where(
                    mc, plsc.bitcast(jnp.bitwise_and(b, himask), jnp.float32),
                    plsc.bitcast(lax.shift_left(b, 16), jnp.float32))
                return acc + av * bv

            acc = lax.fori_loop(0, D, dstep,
                                jnp.zeros((L,), jnp.float32), unroll=16)
            outv[gsl] = acc
            return carry

        lax.fori_loop(0, C // L, group, 0)

    cp_bt.wait()
    cp_bc.wait()

    def addbias(i, carry):
        sl = pl.ds(i * L, L)
        outv[sl] = outv[sl] + btv[sl] + bcv[sl]
        return carry

    lax.fori_loop(0, BPW // L, addbias, 0, unroll=4)

    pltpu.sync_copy(outv, out_hbm.at[pl.ds(base, BPW)])


@jax.jit
def _glove_sc(context_input, target_input, P_context, staged_target,
              b_target_flat, b_context_flat):
    mesh = plsc.VectorSubcoreMesh(core_axis_name="c", subcore_axis_name="s")
    return pl.kernel(
        _glove_body,
        mesh=mesh,
        compiler_params=pltpu.CompilerParams(
            needs_layout_passes=False, use_tc_tiling_on_sc=True),
        out_type=jax.ShapeDtypeStruct((B,), jnp.float32),
        scratch_types=[
            pltpu.VMEM((BPW,), jnp.int32),
            pltpu.VMEM((BPW,), jnp.int32),
            pltpu.VMEM((BPW,), jnp.int32),
            pltpu.VMEM((2, C, 2 * D), jnp.int32),
            pltpu.VMEM((2, C, 2 * D), jnp.int32),
            pltpu.VMEM((BPW,), jnp.float32),
            pltpu.VMEM((BPW,), jnp.float32),
            pltpu.VMEM((BPW,), jnp.float32),
            pltpu.SemaphoreType.DMA,
            pltpu.SemaphoreType.DMA,
            pltpu.SemaphoreType.DMA,
        ],
    )(context_input, target_input, P_context, staged_target,
      b_target_flat, b_context_flat)


def kernel(context_input, target_input, W_target, b_target, W_context,
           b_context):
    ctx = context_input.astype(jnp.int32)
    tgt = target_input.astype(jnp.int32)
    P_t, P_c = _repack2(jnp.transpose(W_target), jnp.transpose(W_context))
    staged = _stage_target(tgt, P_t)
    return _glove_sc(
        ctx,
        tgt,
        P_c,
        staged,
        jnp.reshape(b_target, (V,)),
        jnp.reshape(b_context, (V,)),
    )
